# baseline (speedup 1.0000x reference)
import functools

import jax
import jax.numpy as jnp
from jax import lax
from jax.experimental import pallas as pl
from jax.experimental.pallas import tpu as pltpu
from jax.experimental.pallas import tpu_sc as plsc

NUM_EXPERTS = 8
HIDDEN = 1024
INTER = 512
TOKENS = 2048
TOP_K = 2

TILE = 512
NT = (TOKENS * TOP_K) // TILE + NUM_EXPERTS
P = NT * TILE

NC, NS, L = 2, 16, 16
NW = NC * NS
TPW = TOKENS // NW
CH = 8

_sc_mesh = plsc.VectorSubcoreMesh(core_axis_name="c", subcore_axis_name="s")


@functools.partial(
    pl.kernel,
    mesh=_sc_mesh,
    out_type=jax.ShapeDtypeStruct((P, HIDDEN), jnp.float32),
    scratch_types=[
        pltpu.VMEM((TPW // 2, HIDDEN), jnp.float32),
        pltpu.VMEM((TPW // 2, HIDDEN), jnp.float32),
        pltpu.VMEM((TPW // 2,), jnp.int32),
        pltpu.VMEM((TPW // 2,), jnp.int32),
        pltpu.VMEM((TPW // 2,), jnp.int32),
        pltpu.VMEM((TPW // 2,), jnp.int32),
        pltpu.SemaphoreType.DMA,
        pltpu.SemaphoreType.DMA,
    ],
)
def _dispatch_scatter(x_hbm, row0_hbm, row1_hbm, xs_hbm,
                      xba, xbb, i0a, i0b, i1a, i1b, sem_in, sem_sc):
    wid = lax.axis_index("s") * NC + lax.axis_index("c")
    base = wid * TPW
    half = TPW // 2
    ca = pltpu.async_copy(x_hbm.at[pl.ds(base, half)], xba, sem_in)
    cb = pltpu.async_copy(x_hbm.at[pl.ds(base + half, half)], xbb, sem_in)
    pltpu.sync_copy(row0_hbm.at[pl.ds(base, half)], i0a)
    pltpu.sync_copy(row0_hbm.at[pl.ds(base + half, half)], i0b)
    pltpu.sync_copy(row1_hbm.at[pl.ds(base, half)], i1a)
    pltpu.sync_copy(row1_hbm.at[pl.ds(base + half, half)], i1b)
    ca.wait()
    s0 = pltpu.async_copy(xba, xs_hbm.at[i0a], sem_sc)
    s1 = pltpu.async_copy(xba, xs_hbm.at[i1a], sem_sc)
    cb.wait()
    s2 = pltpu.async_copy(xbb, xs_hbm.at[i0b], sem_sc)
    s3 = pltpu.async_copy(xbb, xs_hbm.at[i1b], sem_sc)
    s0.wait()
    s1.wait()
    s2.wait()
    s3.wait()


def _grouped_mlp_body(pf_ref, xs_ref, g_ref, u_ref, d_ref, ys_ref):
    m = pl.program_id(0)

    @pl.when(pf_ref[2, m] == 1)
    def _():
        x = xs_ref[...]
        g = lax.dot_general(x, g_ref[0], (((1,), (1,)), ((), ())),
                            preferred_element_type=jnp.float32,
                            precision=lax.Precision.DEFAULT)
        u = lax.dot_general(x, u_ref[0], (((1,), (1,)), ((), ())),
                            preferred_element_type=jnp.float32,
                            precision=lax.Precision.DEFAULT)
        h = (g * jax.nn.sigmoid(g)) * u
        ys_ref[...] = lax.dot_general(h, d_ref[0],
                                      (((1,), (1,)), ((), ())),
                                      preferred_element_type=jnp.float32,
                                      precision=lax.Precision.DEFAULT)


def _grouped_mlp(tile_expert, xs, g16, u16, d16):
    grid_spec = pltpu.PrefetchScalarGridSpec(
        num_scalar_prefetch=1,
        grid=(NT,),
        in_specs=[
            pl.BlockSpec((TILE, HIDDEN), lambda m, pf: (pf[1, m], 0)),
            pl.BlockSpec((1, INTER, HIDDEN), lambda m, pf: (pf[0, m], 0, 0)),
            pl.BlockSpec((1, INTER, HIDDEN), lambda m, pf: (pf[0, m], 0, 0)),
            pl.BlockSpec((1, HIDDEN, INTER), lambda m, pf: (pf[0, m], 0, 0)),
        ],
        out_specs=pl.BlockSpec((TILE, HIDDEN), lambda m, pf: (pf[1, m], 0)),
    )
    return pl.pallas_call(
        _grouped_mlp_body,
        grid_spec=grid_spec,
        out_shape=jax.ShapeDtypeStruct((P, HIDDEN), jnp.float32),
        compiler_params=pltpu.CompilerParams(
            dimension_semantics=("arbitrary",),
        ),
    )(tile_expert, xs, g16, u16, d16)


NCH = TPW // CH


@functools.partial(
    pl.kernel,
    mesh=_sc_mesh,
    out_type=jax.ShapeDtypeStruct((TOKENS, HIDDEN), jnp.float32),
    scratch_types=[
        pltpu.VMEM((CH, HIDDEN), jnp.float32),
        pltpu.VMEM((CH, HIDDEN), jnp.float32),
        pltpu.VMEM((CH, HIDDEN), jnp.float32),
        pltpu.VMEM((CH, HIDDEN), jnp.float32),
        pltpu.VMEM((TPW,), jnp.int32),
        pltpu.VMEM((TPW,), jnp.int32),
        pltpu.VMEM((TPW * L,), jnp.float32),
        pltpu.VMEM((TPW * L,), jnp.float32),
        pltpu.SemaphoreType.DMA,
        pltpu.SemaphoreType.DMA,
        pltpu.SemaphoreType.DMA,
    ],
)
def _combine(y_hbm, row0_hbm, row1_hbm, w0_hbm, w1_hbm, out_hbm,
             b0a, b0b, b1a, b1b, idx0, idx1, w0v, w1v, sem_ge, sem_go, sem_st):
    wid = lax.axis_index("s") * NC + lax.axis_index("c")
    base = wid * TPW
    c0 = pltpu.async_copy(row0_hbm.at[pl.ds(base, TPW)], idx0, sem_st)
    c1 = pltpu.async_copy(row1_hbm.at[pl.ds(base, TPW)], idx1, sem_st)
    c2 = pltpu.async_copy(w0_hbm.at[pl.ds(base * L, TPW * L)], w0v, sem_st)
    c3 = pltpu.async_copy(w1_hbm.at[pl.ds(base * L, TPW * L)], w1v, sem_st)
    c0.wait()
    c1.wait()
    c2.wait()
    c3.wait()

    bufs = ((b0a, b0b), (b1a, b1b))
    sems = (sem_ge, sem_go)

    def fire(c):
        ba, bb = bufs[c % 2]
        s = sems[c % 2]
        g0 = pltpu.async_copy(y_hbm.at[idx0.at[pl.ds(c * CH, CH)]], ba, s)
        g1 = pltpu.async_copy(y_hbm.at[idx1.at[pl.ds(c * CH, CH)]], bb, s)
        return g0, g1

    gs = {0: fire(0)}
    sts = {}
    for c in range(NCH):
        if c + 1 < NCH:
            if c - 1 in sts:
                sts[c - 1].wait()
            gs[c + 1] = fire(c + 1)
        g0, g1 = gs[c]
        g0.wait()
        g1.wait()
        ba, bb = bufs[c % 2]

        def row_body(r, carry, c=c, ba=ba, bb=bb):
            off = (c * CH + r) * L
            w0 = w0v[pl.ds(off, L)]
            w1 = w1v[pl.ds(off, L)]
            for j in range(HIDDEN // L):
                a = ba[r, pl.ds(j * L, L)]
                b = bb[r, pl.ds(j * L, L)]
                ba[r, pl.ds(j * L, L)] = a * w0 + b * w1
            return carry

        lax.fori_loop(0, CH, row_body, 0)
        sts[c] = pltpu.async_copy(ba, out_hbm.at[pl.ds(base + c * CH, CH)], sem_st)

    sts[NCH - 2].wait()
    sts[NCH - 1].wait()


_RB = 512
_NB = (TOKENS * TOP_K) // _RB


def _routing_rows(selected_experts):
    e_flat = selected_experts.reshape(-1)
    oh = (e_flat[:, None] == jnp.arange(NUM_EXPERTS, dtype=jnp.int32)[None, :])
    oh = oh.astype(jnp.float32)
    ohb = oh.reshape(_NB, _RB, NUM_EXPERTS)
    tri_inc = jnp.tril(jnp.ones((_RB, _RB), jnp.float32))
    intra = jnp.einsum('ij,bje->bie', tri_inc, ohb)
    bsum = ohb.sum(axis=1)
    tri_exc = jnp.tril(jnp.ones((_NB, _NB), jnp.float32), -1)
    bpre = jnp.einsum('ij,je->ie', tri_exc, bsum)
    pos_incl = (intra + bpre[:, None, :]).reshape(TOKENS * TOP_K, NUM_EXPERTS)
    rank = (oh * pos_incl).sum(axis=1) - 1.0
    counts = bsum.sum(axis=0)
    padded = jnp.floor((counts + (TILE - 1)) / TILE) * TILE
    tri8_exc = jnp.tril(jnp.ones((NUM_EXPERTS, NUM_EXPERTS), jnp.float32), -1)
    pstart = tri8_exc @ padded
    pstart_pair = (oh * pstart[None, :]).sum(axis=1)
    row = (pstart_pair + rank).astype(jnp.int32)
    row2 = row.reshape(TOKENS, TOP_K)
    pend = pstart + padded
    n_active = (pend[-1] / TILE).astype(jnp.int32)
    offs = (jnp.arange(NT, dtype=jnp.float32) * TILE)[:, None]
    offs2 = jnp.minimum(offs, pend[-1] - 1.0)
    te = (offs2 >= pend[None, :]).astype(jnp.int32).sum(axis=1)
    mids = jnp.arange(NT, dtype=jnp.int32)
    xi = jnp.minimum(mids, n_active - 1)
    act = (mids < n_active).astype(jnp.int32)
    prefetch = jnp.stack([te, xi, act])
    return row2[:, 0], row2[:, 1], prefetch


def kernel(hidden_states, routing_weights, selected_experts, gate_proj, up_proj, down_proj):
    sel = selected_experts.astype(jnp.int32)

    row0, row1, tile_expert = _routing_rows(sel)
    w0r = jnp.broadcast_to(routing_weights[:, 0:1], (TOKENS, L)).reshape(-1)
    w1r = jnp.broadcast_to(routing_weights[:, 1:2], (TOKENS, L)).reshape(-1)

    xs = _dispatch_scatter(hidden_states, row0, row1)
    ys = _grouped_mlp(tile_expert, xs, gate_proj, up_proj, down_proj)
    out = _combine(ys, row0, row1, w0r, w1r)
    return out


# speedup vs baseline: 1.2728x; 1.2728x over previous
import functools

import jax
import jax.numpy as jnp
from jax import lax
from jax.experimental import pallas as pl
from jax.experimental.pallas import tpu as pltpu
from jax.experimental.pallas import tpu_sc as plsc

NUM_EXPERTS = 8
HIDDEN = 1024
INTER = 512
TOKENS = 2048
TOP_K = 2

TILE = 576
NT = (TOKENS * TOP_K) // TILE + NUM_EXPERTS
P = NT * TILE

NC, NS, L = 2, 16, 16
NW = NC * NS
TPW = TOKENS // NW
CH = 16

_sc_mesh = plsc.VectorSubcoreMesh(core_axis_name="c", subcore_axis_name="s")


@functools.partial(
    pl.kernel,
    mesh=_sc_mesh,
    out_type=jax.ShapeDtypeStruct((P, HIDDEN), jnp.float32),
    scratch_types=[
        pltpu.VMEM((TPW // 2, HIDDEN), jnp.float32),
        pltpu.VMEM((TPW // 2, HIDDEN), jnp.float32),
        pltpu.VMEM((TPW // 2,), jnp.int32),
        pltpu.VMEM((TPW // 2,), jnp.int32),
        pltpu.VMEM((TPW // 2,), jnp.int32),
        pltpu.VMEM((TPW // 2,), jnp.int32),
        pltpu.SemaphoreType.DMA,
        pltpu.SemaphoreType.DMA,
    ],
)
def _dispatch_scatter(x_hbm, row0_hbm, row1_hbm, xs_hbm,
                      xba, xbb, i0a, i0b, i1a, i1b, sem_in, sem_sc):
    wid = lax.axis_index("s") * NC + lax.axis_index("c")
    base = wid * TPW
    half = TPW // 2
    ca = pltpu.async_copy(x_hbm.at[pl.ds(base, half)], xba, sem_in)
    cb = pltpu.async_copy(x_hbm.at[pl.ds(base + half, half)], xbb, sem_in)
    pltpu.sync_copy(row0_hbm.at[pl.ds(base, half)], i0a)
    pltpu.sync_copy(row0_hbm.at[pl.ds(base + half, half)], i0b)
    pltpu.sync_copy(row1_hbm.at[pl.ds(base, half)], i1a)
    pltpu.sync_copy(row1_hbm.at[pl.ds(base + half, half)], i1b)
    ca.wait()
    s0 = pltpu.async_copy(xba, xs_hbm.at[i0a], sem_sc)
    s1 = pltpu.async_copy(xba, xs_hbm.at[i1a], sem_sc)
    cb.wait()
    s2 = pltpu.async_copy(xbb, xs_hbm.at[i0b], sem_sc)
    s3 = pltpu.async_copy(xbb, xs_hbm.at[i1b], sem_sc)
    s0.wait()
    s1.wait()
    s2.wait()
    s3.wait()


def _grouped_mlp_body(pf_ref, xs_ref, g_ref, u_ref, d_ref, ys_ref):
    m = pl.program_id(0)

    @pl.when(pf_ref[2, m] == 1)
    def _():
        x = xs_ref[...]
        g = lax.dot_general(x, g_ref[0], (((1,), (1,)), ((), ())),
                            preferred_element_type=jnp.float32,
                            precision=lax.Precision.DEFAULT)
        u = lax.dot_general(x, u_ref[0], (((1,), (1,)), ((), ())),
                            preferred_element_type=jnp.float32,
                            precision=lax.Precision.DEFAULT)
        h = (g * jax.nn.sigmoid(g)) * u
        ys_ref[...] = lax.dot_general(h, d_ref[0],
                                      (((1,), (1,)), ((), ())),
                                      preferred_element_type=jnp.float32,
                                      precision=lax.Precision.DEFAULT)


def _grouped_mlp(tile_expert, xs, g16, u16, d16):
    grid_spec = pltpu.PrefetchScalarGridSpec(
        num_scalar_prefetch=1,
        grid=(NT,),
        in_specs=[
            pl.BlockSpec((TILE, HIDDEN), lambda m, pf: (pf[1, m], 0)),
            pl.BlockSpec((1, INTER, HIDDEN), lambda m, pf: (pf[0, m], 0, 0)),
            pl.BlockSpec((1, INTER, HIDDEN), lambda m, pf: (pf[0, m], 0, 0)),
            pl.BlockSpec((1, HIDDEN, INTER), lambda m, pf: (pf[0, m], 0, 0)),
        ],
        out_specs=pl.BlockSpec((TILE, HIDDEN), lambda m, pf: (pf[1, m], 0)),
    )
    return pl.pallas_call(
        _grouped_mlp_body,
        grid_spec=grid_spec,
        out_shape=jax.ShapeDtypeStruct((P, HIDDEN), jnp.float32),
        compiler_params=pltpu.CompilerParams(
            dimension_semantics=("arbitrary",),
        ),
    )(tile_expert, xs, g16, u16, d16)


NCH = TPW // CH


@functools.partial(
    pl.kernel,
    mesh=_sc_mesh,
    out_type=jax.ShapeDtypeStruct((TOKENS, HIDDEN), jnp.float32),
    scratch_types=[
        pltpu.VMEM((CH, HIDDEN), jnp.float32),
        pltpu.VMEM((CH, HIDDEN), jnp.float32),
        pltpu.VMEM((CH, HIDDEN), jnp.float32),
        pltpu.VMEM((CH, HIDDEN), jnp.float32),
        pltpu.VMEM((TPW,), jnp.int32),
        pltpu.VMEM((TPW,), jnp.int32),
        pltpu.VMEM((TPW * L,), jnp.float32),
        pltpu.VMEM((TPW * L,), jnp.float32),
        pltpu.SemaphoreType.DMA,
        pltpu.SemaphoreType.DMA,
        pltpu.SemaphoreType.DMA,
    ],
)
def _combine(y_hbm, row0_hbm, row1_hbm, w0_hbm, w1_hbm, out_hbm,
             b0a, b0b, b1a, b1b, idx0, idx1, w0v, w1v, sem_ge, sem_go, sem_st):
    wid = lax.axis_index("s") * NC + lax.axis_index("c")
    base = wid * TPW
    c0 = pltpu.async_copy(row0_hbm.at[pl.ds(base, TPW)], idx0, sem_st)
    c1 = pltpu.async_copy(row1_hbm.at[pl.ds(base, TPW)], idx1, sem_st)
    c2 = pltpu.async_copy(w0_hbm.at[pl.ds(base * L, TPW * L)], w0v, sem_st)
    c3 = pltpu.async_copy(w1_hbm.at[pl.ds(base * L, TPW * L)], w1v, sem_st)
    c0.wait()
    c1.wait()
    c2.wait()
    c3.wait()

    bufs = ((b0a, b0b), (b1a, b1b))
    sems = (sem_ge, sem_go)

    def fire(c):
        ba, bb = bufs[c % 2]
        s = sems[c % 2]
        g0 = pltpu.async_copy(y_hbm.at[idx0.at[pl.ds(c * CH, CH)]], ba, s)
        g1 = pltpu.async_copy(y_hbm.at[idx1.at[pl.ds(c * CH, CH)]], bb, s)
        return g0, g1

    gs = {0: fire(0)}
    sts = {}
    for c in range(NCH):
        if c + 1 < NCH:
            if c - 1 in sts:
                sts[c - 1].wait()
            gs[c + 1] = fire(c + 1)
        g0, g1 = gs[c]
        g0.wait()
        g1.wait()
        ba, bb = bufs[c % 2]

        def row_body(r, carry, c=c, ba=ba, bb=bb):
            off = (c * CH + r) * L
            w0 = w0v[pl.ds(off, L)]
            w1 = w1v[pl.ds(off, L)]
            for j in range(HIDDEN // L):
                a = ba[r, pl.ds(j * L, L)]
                b = bb[r, pl.ds(j * L, L)]
                ba[r, pl.ds(j * L, L)] = a * w0 + b * w1
            return carry

        lax.fori_loop(0, CH, row_body, 0)
        sts[c] = pltpu.async_copy(ba, out_hbm.at[pl.ds(base + c * CH, CH)], sem_st)

    sts[NCH - 2].wait()
    sts[NCH - 1].wait()


_RB = 512
_NB = (TOKENS * TOP_K) // _RB


def _routing_rows(selected_experts):
    e_flat = selected_experts.reshape(-1)
    oh = (e_flat[:, None] == jnp.arange(NUM_EXPERTS, dtype=jnp.int32)[None, :])
    oh = oh.astype(jnp.float32)
    ohb = oh.reshape(_NB, _RB, NUM_EXPERTS)
    tri_inc = jnp.tril(jnp.ones((_RB, _RB), jnp.float32))
    intra = jnp.einsum('ij,bje->bie', tri_inc, ohb)
    bsum = ohb.sum(axis=1)
    tri_exc = jnp.tril(jnp.ones((_NB, _NB), jnp.float32), -1)
    bpre = jnp.einsum('ij,je->ie', tri_exc, bsum)
    pos_incl = (intra + bpre[:, None, :]).reshape(TOKENS * TOP_K, NUM_EXPERTS)
    rank = (oh * pos_incl).sum(axis=1) - 1.0
    counts = bsum.sum(axis=0)
    padded = jnp.floor((counts + (TILE - 1)) / TILE) * TILE
    tri8_exc = jnp.tril(jnp.ones((NUM_EXPERTS, NUM_EXPERTS), jnp.float32), -1)
    pstart = tri8_exc @ padded
    pstart_pair = (oh * pstart[None, :]).sum(axis=1)
    row = (pstart_pair + rank).astype(jnp.int32)
    row2 = row.reshape(TOKENS, TOP_K)
    pend = pstart + padded
    n_active = (pend[-1] / TILE).astype(jnp.int32)
    offs = (jnp.arange(NT, dtype=jnp.float32) * TILE)[:, None]
    offs2 = jnp.minimum(offs, pend[-1] - 1.0)
    te = (offs2 >= pend[None, :]).astype(jnp.int32).sum(axis=1)
    mids = jnp.arange(NT, dtype=jnp.int32)
    xi = jnp.minimum(mids, n_active - 1)
    act = (mids < n_active).astype(jnp.int32)
    prefetch = jnp.stack([te, xi, act])
    return row2[:, 0], row2[:, 1], prefetch


def kernel(hidden_states, routing_weights, selected_experts, gate_proj, up_proj, down_proj):
    sel = selected_experts.astype(jnp.int32)

    row0, row1, tile_expert = _routing_rows(sel)
    w0r = jnp.broadcast_to(routing_weights[:, 0:1], (TOKENS, L)).reshape(-1)
    w1r = jnp.broadcast_to(routing_weights[:, 1:2], (TOKENS, L)).reshape(-1)

    xs = _dispatch_scatter(hidden_states, row0, row1)
    ys = _grouped_mlp(tile_expert, xs, gate_proj, up_proj, down_proj)
    out = _combine(ys, row0, row1, w0r, w1r)
    return out
